# baseline (speedup 1.0000x reference)
import jax
import jax.numpy as jnp
from jax.experimental import pallas as pl
from jax.experimental.pallas import tpu as pltpu

_PATCH = 8
_GRID_HW = 28
_NUM_PATCHES = _GRID_HW * _GRID_HW
_NUM_KEEP = 196


def _mask_kernel(nrow_ref, ncol_ref, mask_ref, mimg_ref):
    nrow = nrow_ref[0]
    ncol = ncol_ref[0]
    n = _NUM_PATCHES
    jj = jax.lax.broadcasted_iota(jnp.int32, (n, n), 0)
    kk = jax.lax.broadcasted_iota(jnp.int32, (n, n), 1)
    eq = nrow == ncol
    a_row = ((ncol < nrow) | (eq & (jj < kk))).astype(jnp.float32)
    rank_row = jnp.sum(a_row, axis=0, keepdims=True)
    mask_ref[0] = (rank_row < _NUM_KEEP).astype(jnp.float32)
    a_col = ((nrow < ncol) | (eq & (kk < jj))).astype(jnp.float32)
    rank_col = jnp.sum(a_col, axis=1, keepdims=True)
    mask_col = (rank_col < _NUM_KEEP).astype(jnp.float32)
    h = _GRID_HW * _PATCH
    i_f = jax.lax.broadcasted_iota(jnp.int32, (h, n), 0)
    q_f = jax.lax.broadcasted_iota(jnp.int32, (h, n), 1)
    f = (q_f // _GRID_HW == i_f // _PATCH).astype(jnp.float32)
    q_g = jax.lax.broadcasted_iota(jnp.int32, (n, h), 0)
    j_g = jax.lax.broadcasted_iota(jnp.int32, (n, h), 1)
    g = (q_g % _GRID_HW == j_g // _PATCH).astype(jnp.float32)
    mimg_ref[0] = jnp.dot(f, mask_col * g, preferred_element_type=jnp.float32)


def _mul_kernel(x_ref, m_ref, o_ref):
    o_ref[...] = x_ref[...] * m_ref[...]


def kernel(x, noise):
    b, c, h, w = x.shape
    nrow = noise[:, None, :]
    ncol = noise[:, :, None]
    mask3, mimg = pl.pallas_call(
        _mask_kernel,
        grid=(b,),
        in_specs=[
            pl.BlockSpec((1, 1, _NUM_PATCHES), lambda i: (i, 0, 0)),
            pl.BlockSpec((1, _NUM_PATCHES, 1), lambda i: (i, 0, 0)),
        ],
        out_specs=[
            pl.BlockSpec((1, 1, _NUM_PATCHES), lambda i: (i, 0, 0)),
            pl.BlockSpec((1, h, w), lambda i: (i, 0, 0)),
        ],
        out_shape=[
            jax.ShapeDtypeStruct((b, 1, _NUM_PATCHES), jnp.float32),
            jax.ShapeDtypeStruct((b, h, w), jnp.float32),
        ],
    )(nrow, ncol)
    mask = mask3.reshape(b, _NUM_PATCHES)

    rows = (h * w) // 256
    x4 = x.reshape(b, c, rows, 256)
    mi4 = mimg.reshape(b, rows, 256)
    cb = 16
    out = pl.pallas_call(
        _mul_kernel,
        grid=(b, c // cb),
        in_specs=[
            pl.BlockSpec((1, cb, rows, 256), lambda i, j: (i, j, 0, 0)),
            pl.BlockSpec((1, rows, 256), lambda i, j: (i, 0, 0)),
        ],
        out_specs=pl.BlockSpec((1, cb, rows, 256), lambda i, j: (i, j, 0, 0)),
        out_shape=jax.ShapeDtypeStruct((b, c, rows, 256), jnp.float32),
        compiler_params=pltpu.CompilerParams(
            dimension_semantics=("parallel", "parallel")),
    )(x4, mi4)
    return out.reshape(b, c, h, w), mask


# speedup vs baseline: 4.1842x; 4.1842x over previous
import jax
import jax.numpy as jnp
from jax.experimental import pallas as pl
from jax.experimental.pallas import tpu as pltpu

_PATCH = 8
_GRID_HW = 28
_NUM_PATCHES = _GRID_HW * _GRID_HW
_NUM_KEEP = 196


def _mask_kernel(nrow_ref, ncol_ref, mask_ref, mimg_ref):
    nrow = nrow_ref[0]
    ncol = ncol_ref[0]
    n = _NUM_PATCHES
    jj = jax.lax.broadcasted_iota(jnp.int32, (n, n), 0)
    kk = jax.lax.broadcasted_iota(jnp.int32, (n, n), 1)
    eq = nrow == ncol
    a_row = ((ncol < nrow) | (eq & (jj < kk))).astype(jnp.float32)
    rank_row = jnp.sum(a_row, axis=0, keepdims=True)
    mask_ref[0] = (rank_row < _NUM_KEEP).astype(jnp.float32)
    a_col = ((nrow < ncol) | (eq & (kk < jj))).astype(jnp.float32)
    rank_col = jnp.sum(a_col, axis=1, keepdims=True)
    mask_col = (rank_col < _NUM_KEEP).astype(jnp.float32)
    h = _GRID_HW * _PATCH
    i_f = jax.lax.broadcasted_iota(jnp.int32, (h, n), 0)
    q_f = jax.lax.broadcasted_iota(jnp.int32, (h, n), 1)
    f = (q_f // _GRID_HW == i_f // _PATCH).astype(jnp.float32)
    q_g = jax.lax.broadcasted_iota(jnp.int32, (n, h), 0)
    j_g = jax.lax.broadcasted_iota(jnp.int32, (n, h), 1)
    g = (q_g % _GRID_HW == j_g // _PATCH).astype(jnp.float32)
    mimg_ref[0] = jnp.dot(f, mask_col * g, preferred_element_type=jnp.float32)


def _mul_kernel(x_ref, m_ref, o_ref):
    o_ref[...] = x_ref[...] * m_ref[...]


def kernel(x, noise):
    b, c, h, w = x.shape
    nrow = noise[:, None, :]
    ncol = noise[:, :, None]
    mask3, mimg = pl.pallas_call(
        _mask_kernel,
        grid=(b,),
        in_specs=[
            pl.BlockSpec((1, 1, _NUM_PATCHES), lambda i: (i, 0, 0)),
            pl.BlockSpec((1, _NUM_PATCHES, 1), lambda i: (i, 0, 0)),
        ],
        out_specs=[
            pl.BlockSpec((1, 1, _NUM_PATCHES), lambda i: (i, 0, 0)),
            pl.BlockSpec((1, h, w), lambda i: (i, 0, 0)),
        ],
        out_shape=[
            jax.ShapeDtypeStruct((b, 1, _NUM_PATCHES), jnp.float32),
            jax.ShapeDtypeStruct((b, h, w), jnp.float32),
        ],
    )(nrow, ncol)
    mask = mask3.reshape(b, _NUM_PATCHES)

    cb = 16
    out = pl.pallas_call(
        _mul_kernel,
        grid=(b, c // cb),
        in_specs=[
            pl.BlockSpec((1, cb, h, w), lambda i, j: (i, j, 0, 0)),
            pl.BlockSpec((1, h, w), lambda i, j: (i, 0, 0)),
        ],
        out_specs=pl.BlockSpec((1, cb, h, w), lambda i, j: (i, j, 0, 0)),
        out_shape=jax.ShapeDtypeStruct((b, c, h, w), jnp.float32),
        compiler_params=pltpu.CompilerParams(
            dimension_semantics=("parallel", "parallel")),
    )(x, mimg)
    return out, mask


# speedup vs baseline: 4.2680x; 1.0200x over previous
import jax
import jax.numpy as jnp
from jax.experimental import pallas as pl
from jax.experimental.pallas import tpu as pltpu

_PATCH = 8
_GRID_HW = 28
_NUM_PATCHES = _GRID_HW * _GRID_HW
_NUM_KEEP = 196


def _mask_kernel(nrow_ref, ncol_ref, mask_ref, mimg_ref):
    nrow = nrow_ref[0]
    ncol = ncol_ref[0]
    n = _NUM_PATCHES
    jj = jax.lax.broadcasted_iota(jnp.int32, (n, n), 0)
    kk = jax.lax.broadcasted_iota(jnp.int32, (n, n), 1)
    eq = nrow == ncol
    a_row = ((ncol < nrow) | (eq & (jj < kk))).astype(jnp.float32)
    rank_row = jnp.sum(a_row, axis=0, keepdims=True)
    mask_ref[0] = (rank_row < _NUM_KEEP).astype(jnp.float32)
    a_col = ((nrow < ncol) | (eq & (kk < jj))).astype(jnp.float32)
    rank_col = jnp.sum(a_col, axis=1, keepdims=True)
    mask_col = (rank_col < _NUM_KEEP).astype(jnp.float32)
    h = _GRID_HW * _PATCH
    i_f = jax.lax.broadcasted_iota(jnp.int32, (h, n), 0)
    q_f = jax.lax.broadcasted_iota(jnp.int32, (h, n), 1)
    f = (q_f // _GRID_HW == i_f // _PATCH).astype(jnp.float32)
    q_g = jax.lax.broadcasted_iota(jnp.int32, (n, h), 0)
    j_g = jax.lax.broadcasted_iota(jnp.int32, (n, h), 1)
    g = (q_g % _GRID_HW == j_g // _PATCH).astype(jnp.float32)
    mimg_ref[0] = jnp.dot(f, mask_col * g, preferred_element_type=jnp.float32)


def _mul_kernel(x_ref, m_ref, o_ref):
    o_ref[...] = x_ref[...] * m_ref[...]


def kernel(x, noise):
    b, c, h, w = x.shape
    nrow = noise[:, None, :]
    ncol = noise[:, :, None]
    mask3, mimg = pl.pallas_call(
        _mask_kernel,
        grid=(b,),
        in_specs=[
            pl.BlockSpec((1, 1, _NUM_PATCHES), lambda i: (i, 0, 0)),
            pl.BlockSpec((1, _NUM_PATCHES, 1), lambda i: (i, 0, 0)),
        ],
        out_specs=[
            pl.BlockSpec((1, 1, _NUM_PATCHES), lambda i: (i, 0, 0)),
            pl.BlockSpec((1, h, w), lambda i: (i, 0, 0)),
        ],
        out_shape=[
            jax.ShapeDtypeStruct((b, 1, _NUM_PATCHES), jnp.float32),
            jax.ShapeDtypeStruct((b, h, w), jnp.float32),
        ],
    )(nrow, ncol)
    mask = mask3.reshape(b, _NUM_PATCHES)

    cb = 32
    out = pl.pallas_call(
        _mul_kernel,
        grid=(b, c // cb),
        in_specs=[
            pl.BlockSpec((1, cb, h, w), lambda i, j: (i, j, 0, 0)),
            pl.BlockSpec((1, h, w), lambda i, j: (i, 0, 0)),
        ],
        out_specs=pl.BlockSpec((1, cb, h, w), lambda i, j: (i, j, 0, 0)),
        out_shape=jax.ShapeDtypeStruct((b, c, h, w), jnp.float32),
        compiler_params=pltpu.CompilerParams(
            dimension_semantics=("parallel", "parallel")),
    )(x, mimg)
    return out, mask


# speedup vs baseline: 4.4559x; 1.0440x over previous
import jax
import jax.numpy as jnp
from jax.experimental import pallas as pl
from jax.experimental.pallas import tpu as pltpu

_PATCH = 8
_GRID_HW = 28
_NUM_PATCHES = _GRID_HW * _GRID_HW
_NUM_KEEP = 196


def _fused_kernel(nrow_ref, ncol_ref, x_ref, o_ref, mask_ref, mimg_ref):
    j = pl.program_id(1)

    @pl.when(j == 0)
    def _():
        nrow = nrow_ref[0]
        ncol = ncol_ref[0]
        n = _NUM_PATCHES
        jj = jax.lax.broadcasted_iota(jnp.int32, (n, n), 0)
        kk = jax.lax.broadcasted_iota(jnp.int32, (n, n), 1)
        eq = nrow == ncol
        a_row = ((ncol < nrow) | (eq & (jj < kk))).astype(jnp.float32)
        rank_row = jnp.sum(a_row, axis=0, keepdims=True)
        mask_ref[0] = (rank_row < _NUM_KEEP).astype(jnp.float32)
        a_col = ((nrow < ncol) | (eq & (kk < jj))).astype(jnp.float32)
        rank_col = jnp.sum(a_col, axis=1, keepdims=True)
        mask_col = (rank_col < _NUM_KEEP).astype(jnp.float32)
        h = _GRID_HW * _PATCH
        i_f = jax.lax.broadcasted_iota(jnp.int32, (h, n), 0)
        q_f = jax.lax.broadcasted_iota(jnp.int32, (h, n), 1)
        f = (q_f // _GRID_HW == i_f // _PATCH).astype(jnp.float32)
        q_g = jax.lax.broadcasted_iota(jnp.int32, (n, h), 0)
        j_g = jax.lax.broadcasted_iota(jnp.int32, (n, h), 1)
        g = (q_g % _GRID_HW == j_g // _PATCH).astype(jnp.float32)
        mimg_ref[...] = jnp.dot(f, mask_col * g,
                                preferred_element_type=jnp.float32)

    o_ref[...] = x_ref[...] * mimg_ref[...]


def kernel(x, noise):
    b, c, h, w = x.shape
    nrow = noise[:, None, :]
    ncol = noise[:, :, None]
    cb = 32
    out, mask3 = pl.pallas_call(
        _fused_kernel,
        grid=(b, c // cb),
        in_specs=[
            pl.BlockSpec((1, 1, _NUM_PATCHES), lambda i, j: (i, 0, 0)),
            pl.BlockSpec((1, _NUM_PATCHES, 1), lambda i, j: (i, 0, 0)),
            pl.BlockSpec((1, cb, h, w), lambda i, j: (i, j, 0, 0)),
        ],
        out_specs=[
            pl.BlockSpec((1, cb, h, w), lambda i, j: (i, j, 0, 0)),
            pl.BlockSpec((1, 1, _NUM_PATCHES), lambda i, j: (i, 0, 0)),
        ],
        out_shape=[
            jax.ShapeDtypeStruct((b, c, h, w), jnp.float32),
            jax.ShapeDtypeStruct((b, 1, _NUM_PATCHES), jnp.float32),
        ],
        scratch_shapes=[pltpu.VMEM((h, w), jnp.float32)],
        compiler_params=pltpu.CompilerParams(
            dimension_semantics=("arbitrary", "arbitrary")),
    )(nrow, ncol, x)
    return out, mask3.reshape(b, _NUM_PATCHES)


# speedup vs baseline: 4.5090x; 1.0119x over previous
import jax
import jax.numpy as jnp
from jax.experimental import pallas as pl
from jax.experimental.pallas import tpu as pltpu

_PATCH = 8
_GRID_HW = 28
_NUM_PATCHES = _GRID_HW * _GRID_HW
_NUM_KEEP = 196


def _fused_kernel(nrow_ref, ncol_ref, x_ref, o_ref, mask_ref, mimg_ref):
    j = pl.program_id(1)

    @pl.when(j == 0)
    def _():
        nrow = nrow_ref[0]
        ncol = ncol_ref[0]
        n = _NUM_PATCHES
        jj = jax.lax.broadcasted_iota(jnp.int32, (n, n), 0)
        kk = jax.lax.broadcasted_iota(jnp.int32, (n, n), 1)
        eq = nrow == ncol
        a_row = ((ncol < nrow) | (eq & (jj < kk))).astype(jnp.float32)
        rank_row = jnp.sum(a_row, axis=0, keepdims=True)
        mask_ref[0] = (rank_row < _NUM_KEEP).astype(jnp.float32)
        a_col = ((nrow < ncol) | (eq & (kk < jj))).astype(jnp.float32)
        rank_col = jnp.sum(a_col, axis=1, keepdims=True)
        mask_col = (rank_col < _NUM_KEEP).astype(jnp.float32)
        h = _GRID_HW * _PATCH
        i_f = jax.lax.broadcasted_iota(jnp.int32, (h, n), 0)
        q_f = jax.lax.broadcasted_iota(jnp.int32, (h, n), 1)
        f = (q_f // _GRID_HW == i_f // _PATCH).astype(jnp.float32)
        q_g = jax.lax.broadcasted_iota(jnp.int32, (n, h), 0)
        j_g = jax.lax.broadcasted_iota(jnp.int32, (n, h), 1)
        g = (q_g % _GRID_HW == j_g // _PATCH).astype(jnp.float32)
        mimg_ref[...] = jnp.dot(f, mask_col * g,
                                preferred_element_type=jnp.float32)

    o_ref[...] = x_ref[...] * mimg_ref[...]


def kernel(x, noise):
    b, c, h, w = x.shape
    nrow = noise[:, None, :]
    ncol = noise[:, :, None]
    cb = 48
    out, mask3 = pl.pallas_call(
        _fused_kernel,
        grid=(b, c // cb),
        in_specs=[
            pl.BlockSpec((1, 1, _NUM_PATCHES), lambda i, j: (i, 0, 0)),
            pl.BlockSpec((1, _NUM_PATCHES, 1), lambda i, j: (i, 0, 0)),
            pl.BlockSpec((1, cb, h, w), lambda i, j: (i, j, 0, 0)),
        ],
        out_specs=[
            pl.BlockSpec((1, cb, h, w), lambda i, j: (i, j, 0, 0)),
            pl.BlockSpec((1, 1, _NUM_PATCHES), lambda i, j: (i, 0, 0)),
        ],
        out_shape=[
            jax.ShapeDtypeStruct((b, c, h, w), jnp.float32),
            jax.ShapeDtypeStruct((b, 1, _NUM_PATCHES), jnp.float32),
        ],
        scratch_shapes=[pltpu.VMEM((h, w), jnp.float32)],
        compiler_params=pltpu.CompilerParams(
            dimension_semantics=("arbitrary", "arbitrary")),
    )(nrow, ncol, x)
    return out, mask3.reshape(b, _NUM_PATCHES)


# speedup vs baseline: 4.5521x; 1.0096x over previous
import jax
import jax.numpy as jnp
from jax.experimental import pallas as pl
from jax.experimental.pallas import tpu as pltpu

_PATCH = 8
_GRID_HW = 28
_NUM_PATCHES = _GRID_HW * _GRID_HW
_NUM_KEEP = 196


def _fused_kernel(nrow_ref, ncol_ref, x_ref, o_ref, mask_ref, mimg_ref):
    j = pl.program_id(1)

    @pl.when(j == 0)
    def _():
        nrow = nrow_ref[0]
        ncol = ncol_ref[0]
        n = _NUM_PATCHES
        jj = jax.lax.broadcasted_iota(jnp.int32, (n, n), 0)
        kk = jax.lax.broadcasted_iota(jnp.int32, (n, n), 1)
        eq = nrow == ncol
        a_row = ((ncol < nrow) | (eq & (jj < kk))).astype(jnp.float32)
        rank_row = jnp.sum(a_row, axis=0, keepdims=True)
        mask_ref[0] = (rank_row < _NUM_KEEP).astype(jnp.float32)
        a_col = ((nrow < ncol) | (eq & (kk < jj))).astype(jnp.float32)
        rank_col = jnp.sum(a_col, axis=1, keepdims=True)
        mask_col = (rank_col < _NUM_KEEP).astype(jnp.float32)
        h = _GRID_HW * _PATCH
        i_f = jax.lax.broadcasted_iota(jnp.int32, (h, n), 0)
        q_f = jax.lax.broadcasted_iota(jnp.int32, (h, n), 1)
        f = (q_f // _GRID_HW == i_f // _PATCH).astype(jnp.float32)
        q_g = jax.lax.broadcasted_iota(jnp.int32, (n, h), 0)
        j_g = jax.lax.broadcasted_iota(jnp.int32, (n, h), 1)
        g = (q_g % _GRID_HW == j_g // _PATCH).astype(jnp.float32)
        mimg_ref[...] = jnp.dot(f, mask_col * g,
                                preferred_element_type=jnp.float32)

    o_ref[...] = x_ref[...] * mimg_ref[...]


def kernel(x, noise):
    b, c, h, w = x.shape
    nrow = noise[:, None, :]
    ncol = noise[:, :, None]
    cb = 64
    out, mask3 = pl.pallas_call(
        _fused_kernel,
        grid=(b, c // cb),
        in_specs=[
            pl.BlockSpec((1, 1, _NUM_PATCHES), lambda i, j: (i, 0, 0)),
            pl.BlockSpec((1, _NUM_PATCHES, 1), lambda i, j: (i, 0, 0)),
            pl.BlockSpec((1, cb, h, w), lambda i, j: (i, j, 0, 0)),
        ],
        out_specs=[
            pl.BlockSpec((1, cb, h, w), lambda i, j: (i, j, 0, 0)),
            pl.BlockSpec((1, 1, _NUM_PATCHES), lambda i, j: (i, 0, 0)),
        ],
        out_shape=[
            jax.ShapeDtypeStruct((b, c, h, w), jnp.float32),
            jax.ShapeDtypeStruct((b, 1, _NUM_PATCHES), jnp.float32),
        ],
        scratch_shapes=[pltpu.VMEM((h, w), jnp.float32)],
        compiler_params=pltpu.CompilerParams(
            dimension_semantics=("arbitrary", "arbitrary")),
    )(nrow, ncol, x)
    return out, mask3.reshape(b, _NUM_PATCHES)
